# speedup vs baseline: 1.0104x; 1.0104x over previous
import math
import functools

import jax
import jax.numpy as jnp
from jax import lax
from jax.experimental import pallas as pl

R, N = 128, 32768
K = math.floor(0.1 * N)


def _prob_body(x_ref, p_ref):
    x = x_ref[...]
    a = jnp.abs(x)
    m = jnp.max(a, axis=1, keepdims=True)
    p_ref[...] = a / m


def _compute_prob(x):
    return pl.pallas_call(
        _prob_body,
        grid=(R // 8,),
        in_specs=[pl.BlockSpec((8, N), lambda i: (i, 0))],
        out_specs=pl.BlockSpec((8, N), lambda i: (i, 0)),
        out_shape=jax.ShapeDtypeStruct((R, N), jnp.float32),
    )(x)


def _rotl(v, d):
    u = jnp.uint32(d)
    return (v << u) | (v >> jnp.uint32(32 - d))


def _mask_body(x_ref, p_ref, o_ref, *, block_cols):
    i = pl.program_id(0)
    x = x_ref[...]
    p = p_ref[...]
    rows_blk, cols_blk = x.shape
    row = lax.broadcasted_iota(jnp.uint32, (rows_blk, cols_blk), 0)
    col = lax.broadcasted_iota(jnp.uint32, (rows_blk, cols_blk), 1)
    n = row * jnp.uint32(N) + col + jnp.uint32(block_cols) * i.astype(jnp.uint32)
    ks0 = jnp.uint32(0)
    ks1 = jnp.uint32(42)
    ks2 = jnp.uint32(42 ^ 0x1BD11BDA)
    x0 = jnp.full_like(n, ks0)
    x1 = n + ks1

    def rounds(x0, x1, rots):
        for r in rots:
            x0 = x0 + x1
            x1 = _rotl(x1, r)
            x1 = x0 ^ x1
        return x0, x1

    ra = (13, 15, 26, 6)
    rb = (17, 29, 16, 24)
    x0, x1 = rounds(x0, x1, ra)
    x0 += ks1
    x1 += ks2 + jnp.uint32(1)
    x0, x1 = rounds(x0, x1, rb)
    x0 += ks2
    x1 += ks0 + jnp.uint32(2)
    x0, x1 = rounds(x0, x1, ra)
    x0 += ks0
    x1 += ks1 + jnp.uint32(3)
    x0, x1 = rounds(x0, x1, rb)
    x0 += ks1
    x1 += ks2 + jnp.uint32(4)
    x0, x1 = rounds(x0, x1, ra)
    x0 += ks2
    x1 += ks0 + jnp.uint32(5)
    bits = x0 ^ x1

    fb = (bits >> jnp.uint32(9)) | jnp.uint32(0x3F800000)
    u = lax.bitcast_convert_type(fb, jnp.float32) - jnp.float32(1.0)
    keep = u < (jnp.float32(1.0) - p)
    o_ref[...] = jnp.where(keep, x, jnp.float32(0.0))


def _apply_mask(x, prob):
    block_cols = 4096
    return pl.pallas_call(
        functools.partial(_mask_body, block_cols=block_cols),
        grid=(N // block_cols,),
        in_specs=[
            pl.BlockSpec((R, block_cols), lambda i: (0, i)),
            pl.BlockSpec((R, block_cols), lambda i: (0, i)),
        ],
        out_specs=pl.BlockSpec((R, block_cols), lambda i: (0, i)),
        out_shape=jax.ShapeDtypeStruct((R, N), jnp.float32),
    )(x, prob)


def _topk_replace_xla(prob):
    top_g, top_idx = lax.top_k(prob, K)
    neg_vals, btm_idx = lax.top_k(-prob, K)
    btm_g = jnp.take_along_axis(prob, btm_idx, axis=1)
    scalling = top_g - (top_g - btm_g)
    rows = jnp.arange(R)[:, None]
    return prob.at[rows, top_idx].set(scalling)


def kernel(x):
    prob = _compute_prob(x)
    prob = _topk_replace_xla(prob)
    return _apply_mask(x, prob)


# speedup vs baseline: 8.6853x; 8.5961x over previous
import math
import functools

import jax
import jax.numpy as jnp
from jax import lax
from jax.experimental import pallas as pl
from jax.experimental.pallas import tpu as pltpu
from jax.experimental.pallas import tpu_sc as plsc

R, N = 128, 32768
K = math.floor(0.1 * N)
NVEC = N // 16
CAP = 8192
NW = 32
ROWS_PER_W = R // NW
SENT_HI = 0x7FFFFFFF


def _sc_body(x_hbm, out_hbm, P, h1, h2, BA, BB, TA, TB, IA, IB):
    wid = lax.axis_index("s") * 2 + lax.axis_index("c")
    lane = lax.iota(jnp.int32, 16)
    zeros16 = jnp.zeros((16,), jnp.int32)

    czero, _ = plsc.scan_count(zeros16)
    c0 = jnp.min(czero)
    e0 = jnp.int32(1) - c0

    def hist_bump(href, d, cnt, lastm):
        base = plsc.load_gather(href, [d])
        plsc.store_scatter(href, [d], base + cnt + e0, mask=lastm)
        return base

    def clear(href, nv):
        def body(i, _):
            href[pl.ds(i * 16, 16)] = zeros16
            return 0
        lax.fori_loop(0, nv, body, 0)

    def radix_pass(src, dst, shift, nvec, isrc, idst):
        clear(h2, 64)

        def hist(i, _):
            u = src[pl.ds(i * 16, 16)]
            d = (u >> shift) & 1023
            cnt, lastm = plsc.scan_count(d)
            hist_bump(h2, d, cnt, lastm)
            return 0
        lax.fori_loop(0, nvec, hist, 0)

        def csum(i, carry):
            v = h2[pl.ds(i * 16, 16)]
            s = plsc.cumsum(v)
            h2[pl.ds(i * 16, 16)] = s - v + carry
            return carry + jnp.max(s)
        lax.fori_loop(0, 64, csum, jnp.int32(0))

        def scat(i, _):
            u = src[pl.ds(i * 16, 16)]
            d = (u >> shift) & 1023
            cnt, lastm = plsc.scan_count(d)
            base = hist_bump(h2, d, cnt, lastm)
            off = base + cnt - c0
            plsc.store_scatter(dst, [off], u)
            if isrc is not None:
                ix = isrc[pl.ds(i * 16, 16)]
                plsc.store_scatter(idst, [off], ix)
            return 0
        lax.fori_loop(0, nvec, scat, 0)

    def row_body(rr, _):
        row = wid * ROWS_PER_W + rr
        pltpu.sync_copy(x_hbm.at[row], P)

        def mx(i, acc):
            return jnp.maximum(acc, jnp.abs(P[pl.ds(i * 16, 16)]))
        macc = lax.fori_loop(0, NVEC, mx, jnp.zeros((16,), jnp.float32))
        m = jnp.max(macc)

        clear(h1, 256)

        def ph(i, _):
            sl = pl.ds(i * 16, 16)
            p = jnp.abs(P[sl]) / m
            P[sl] = p
            d = plsc.bitcast(p, jnp.int32) >> 18
            cnt, lastm = plsc.scan_count(d)
            hist_bump(h1, d, cnt, lastm)
            return 0
        lax.fori_loop(0, NVEC, ph, 0)

        def cs1(i, carry):
            c, t1, t2 = carry
            v = h1[pl.ds(i * 16, 16)]
            s = plsc.cumsum(v)
            ex = s - v + c
            h1[pl.ds(i * 16, 16)] = ex
            t1 = t1 + jnp.max(plsc.all_reduce_population_count(ex < K))
            t2 = t2 + jnp.max(plsc.all_reduce_population_count(ex <= N - K))
            return (c + jnp.max(s), t1, t2)
        _, t1, t2 = lax.fori_loop(
            0, 256, cs1, (jnp.int32(0), jnp.int32(0), jnp.int32(0)))
        H = t2 - 1

        def cp(i, carry):
            pb, pt = carry
            sl = pl.ds(i * 16, 16)
            u = plsc.bitcast(P[sl], jnp.int32)
            d = u >> 18
            mB = d < t1
            mT = d >= H
            plsc.store_compressed(BA.at[pl.ds(pb, 16)], u, mask=mB)
            plsc.store_compressed(TA.at[pl.ds(pt, 16)], u, mask=mT)
            plsc.store_compressed(IA.at[pl.ds(pt, 16)], lane + i * 16, mask=mT)
            pb = pb + jnp.max(plsc.all_reduce_population_count(mB))
            pt = pt + jnp.max(plsc.all_reduce_population_count(mT))
            return (pb, pt)
        posB, posT = lax.fori_loop(0, NVEC, cp, (jnp.int32(0), jnp.int32(0)))

        BA[pl.ds(posB, 16)] = jnp.full((16,), SENT_HI, jnp.int32)
        TA[pl.ds(posT, 16)] = zeros16
        IA[pl.ds(posT, 16)] = zeros16
        nbB = (posB + 15) >> 4
        nbT = (posT + 15) >> 4
        STp = nbT * 16

        radix_pass(BA, BB, 0, nbB, None, None)
        radix_pass(BB, BA, 10, nbB, None, None)
        radix_pass(BA, BB, 20, nbB, None, None)

        radix_pass(TA, TB, 0, nbT, IA, IB)
        radix_pass(TB, TA, 10, nbT, IB, IA)
        radix_pass(TA, TB, 20, nbT, IA, IB)

        def rep(i, _):
            t = jnp.minimum(lane + i * 16, K - 1)
            j = STp - K + t
            vu = plsc.load_gather(TB, [j])
            ti = plsc.load_gather(IB, [j])
            bu = plsc.load_gather(BB, [K - 1 - t])
            v = plsc.bitcast(vu, jnp.float32)
            b = plsc.bitcast(bu, jnp.float32)
            plsc.store_scatter(P, [ti], v - (v - b))
            return 0
        lax.fori_loop(0, (K + 15) // 16, rep, 0)

        pltpu.sync_copy(P, out_hbm.at[row])
        return 0

    lax.fori_loop(0, ROWS_PER_W, row_body, 0)


@functools.partial(jax.jit, static_argnums=())
def _sc_topk_replace(x):
    kfn = pl.kernel(
        _sc_body,
        out_type=jax.ShapeDtypeStruct((R, N), jnp.float32),
        mesh=plsc.VectorSubcoreMesh(core_axis_name="c", subcore_axis_name="s"),
        compiler_params=pltpu.CompilerParams(needs_layout_passes=False),
        scratch_types=[
            pltpu.VMEM((N,), jnp.float32),
            pltpu.VMEM((4096,), jnp.int32),
            pltpu.VMEM((1024,), jnp.int32),
            pltpu.VMEM((CAP,), jnp.int32),
            pltpu.VMEM((CAP,), jnp.int32),
            pltpu.VMEM((CAP,), jnp.int32),
            pltpu.VMEM((CAP,), jnp.int32),
            pltpu.VMEM((CAP,), jnp.int32),
            pltpu.VMEM((CAP,), jnp.int32),
        ],
    )
    return kfn(x)


def _rotl(v, d):
    u = jnp.uint32(d)
    return (v << u) | (v >> jnp.uint32(32 - d))


def _mask_body(x_ref, p_ref, o_ref, *, block_cols):
    i = pl.program_id(0)
    x = x_ref[...]
    p = p_ref[...]
    rows_blk, cols_blk = x.shape
    row = lax.broadcasted_iota(jnp.uint32, (rows_blk, cols_blk), 0)
    col = lax.broadcasted_iota(jnp.uint32, (rows_blk, cols_blk), 1)
    n = row * jnp.uint32(N) + col + jnp.uint32(block_cols) * i.astype(jnp.uint32)
    ks0 = jnp.uint32(0)
    ks1 = jnp.uint32(42)
    ks2 = jnp.uint32(42 ^ 0x1BD11BDA)
    x0 = jnp.full_like(n, ks0)
    x1 = n + ks1

    def rounds(x0, x1, rots):
        for r in rots:
            x0 = x0 + x1
            x1 = _rotl(x1, r)
            x1 = x0 ^ x1
        return x0, x1

    ra = (13, 15, 26, 6)
    rb = (17, 29, 16, 24)
    x0, x1 = rounds(x0, x1, ra)
    x0 += ks1
    x1 += ks2 + jnp.uint32(1)
    x0, x1 = rounds(x0, x1, rb)
    x0 += ks2
    x1 += ks0 + jnp.uint32(2)
    x0, x1 = rounds(x0, x1, ra)
    x0 += ks0
    x1 += ks1 + jnp.uint32(3)
    x0, x1 = rounds(x0, x1, rb)
    x0 += ks1
    x1 += ks2 + jnp.uint32(4)
    x0, x1 = rounds(x0, x1, ra)
    x0 += ks2
    x1 += ks0 + jnp.uint32(5)
    bits = x0 ^ x1

    fb = (bits >> jnp.uint32(9)) | jnp.uint32(0x3F800000)
    u = lax.bitcast_convert_type(fb, jnp.float32) - jnp.float32(1.0)
    keep = u < (jnp.float32(1.0) - p)
    o_ref[...] = jnp.where(keep, x, jnp.float32(0.0))


def _apply_mask(x, prob):
    block_cols = 4096
    return pl.pallas_call(
        functools.partial(_mask_body, block_cols=block_cols),
        grid=(N // block_cols,),
        in_specs=[
            pl.BlockSpec((R, block_cols), lambda i: (0, i)),
            pl.BlockSpec((R, block_cols), lambda i: (0, i)),
        ],
        out_specs=pl.BlockSpec((R, block_cols), lambda i: (0, i)),
        out_shape=jax.ShapeDtypeStruct((R, N), jnp.float32),
    )(x, prob)


def kernel(x):
    new_prob = _sc_topk_replace(x)
    return _apply_mask(x, new_prob)


# speedup vs baseline: 10.4576x; 1.2041x over previous
import math
import functools

import jax
import jax.numpy as jnp
from jax import lax
from jax.experimental import pallas as pl
from jax.experimental.pallas import tpu as pltpu
from jax.experimental.pallas import tpu_sc as plsc

R, N = 128, 32768
K = math.floor(0.1 * N)
NVEC = N // 16
CAP = 8192
NW = 32
ROWS_PER_W = R // NW
SENT_HI = 0x7FFFFFFF


def _lane0(v):
    return lax.squeeze(lax.slice(v, (0,), (1,)), (0,))


def _lane15(v):
    return lax.squeeze(lax.slice(v, (15,), (16,)), (0,))


def _sc_body(x_hbm, out_hbm, P, h1, h2, BA, BB, TA, TB, IA, IB):
    wid = lax.axis_index("s") * 2 + lax.axis_index("c")
    lane = lax.iota(jnp.int32, 16)
    zeros16 = jnp.zeros((16,), jnp.int32)

    czero, _ = plsc.scan_count(zeros16)
    c0 = jnp.min(czero)
    e0 = jnp.int32(1) - c0

    def hist_bump(href, d, cnt, lastm):
        base = plsc.load_gather(href, [d])
        plsc.store_scatter(href, [d], base + cnt + e0, mask=lastm)
        return base

    def clear(href, nv):
        def body(i, _):
            href[pl.ds(i * 16, 16)] = zeros16
            return 0
        lax.fori_loop(0, nv, body, 0)

    def radix_pass(src, dst, shift, nvec, isrc, idst):
        clear(h2, 64)

        def hist(i, _):
            u = src[pl.ds(i * 16, 16)]
            d = (u >> shift) & 1023
            cnt, lastm = plsc.scan_count(d)
            hist_bump(h2, d, cnt, lastm)
            return 0
        lax.fori_loop(0, nvec, hist, 0)

        def csum(i, carry):
            v = h2[pl.ds(i * 16, 16)]
            s = plsc.cumsum(v)
            h2[pl.ds(i * 16, 16)] = s - v + carry
            return carry + _lane15(s)
        lax.fori_loop(0, 64, csum, jnp.int32(0))

        def scat(i, _):
            u = src[pl.ds(i * 16, 16)]
            d = (u >> shift) & 1023
            cnt, lastm = plsc.scan_count(d)
            base = hist_bump(h2, d, cnt, lastm)
            off = base + cnt - c0
            plsc.store_scatter(dst, [off], u)
            if isrc is not None:
                ix = isrc[pl.ds(i * 16, 16)]
                plsc.store_scatter(idst, [off], ix)
            return 0
        lax.fori_loop(0, nvec, scat, 0)

    def row_body(rr, _):
        row = wid * ROWS_PER_W + rr
        pltpu.sync_copy(x_hbm.at[row], P)

        clear(h1, 256)

        def mh(i, acc):
            a = jnp.abs(P[pl.ds(i * 16, 16)])
            d = plsc.bitcast(a, jnp.int32) >> 19
            cnt, lastm = plsc.scan_count(d)
            hist_bump(h1, d, cnt, lastm)
            return jnp.maximum(acc, a)
        macc = lax.fori_loop(0, NVEC, mh, jnp.zeros((16,), jnp.float32))
        m = jnp.max(macc)

        def cs1(i, carry):
            c, t1, t2 = carry
            v = h1[pl.ds(i * 16, 16)]
            s = plsc.cumsum(v)
            ex = s - v + c
            t1 = t1 + _lane0(plsc.all_reduce_population_count(ex < K))
            t2 = t2 + _lane0(plsc.all_reduce_population_count(ex <= N - K))
            return (c + _lane15(s), t1, t2)
        _, t1, t2 = lax.fori_loop(
            0, 256, cs1, (jnp.int32(0), jnp.int32(0), jnp.int32(0)))
        H = t2 - 1

        def cp(i, carry):
            pb, pt = carry
            sl = pl.ds(i * 16, 16)
            a = jnp.abs(P[sl])
            d = plsc.bitcast(a, jnp.int32) >> 19
            p = a / m
            P[sl] = p
            u = plsc.bitcast(p, jnp.int32)
            mB = d < t1
            mT = d >= H
            plsc.store_compressed(BA.at[pl.ds(pb, 16)], u, mask=mB)
            plsc.store_compressed(TA.at[pl.ds(pt, 16)], u, mask=mT)
            plsc.store_compressed(IA.at[pl.ds(pt, 16)], lane + i * 16, mask=mT)
            pb = pb + _lane0(plsc.all_reduce_population_count(mB))
            pt = pt + _lane0(plsc.all_reduce_population_count(mT))
            return (pb, pt)
        posB, posT = lax.fori_loop(0, NVEC, cp, (jnp.int32(0), jnp.int32(0)))

        BA[pl.ds(posB, 16)] = jnp.full((16,), SENT_HI, jnp.int32)
        TA[pl.ds(posT, 16)] = zeros16
        IA[pl.ds(posT, 16)] = zeros16
        nbB = (posB + 15) >> 4
        nbT = (posT + 15) >> 4
        STp = nbT * 16

        radix_pass(BA, BB, 10, nbB, None, None)
        radix_pass(BB, BA, 20, nbB, None, None)

        radix_pass(TA, TB, 0, nbT, IA, IB)
        radix_pass(TB, TA, 10, nbT, IB, IA)
        radix_pass(TA, TB, 20, nbT, IA, IB)

        def rep(i, _):
            t = jnp.minimum(lane + i * 16, K - 1)
            j = STp - K + t
            vu = plsc.load_gather(TB, [j])
            ti = plsc.load_gather(IB, [j])
            bu = plsc.load_gather(BA, [K - 1 - t])
            v = plsc.bitcast(vu, jnp.float32)
            b = plsc.bitcast(bu, jnp.float32)
            plsc.store_scatter(P, [ti], v - (v - b))
            return 0
        lax.fori_loop(0, (K + 15) // 16, rep, 0)

        pltpu.sync_copy(P, out_hbm.at[row])
        return 0

    lax.fori_loop(0, ROWS_PER_W, row_body, 0)


@functools.partial(jax.jit, static_argnums=())
def _sc_topk_replace(x):
    kfn = pl.kernel(
        _sc_body,
        out_type=jax.ShapeDtypeStruct((R, N), jnp.float32),
        mesh=plsc.VectorSubcoreMesh(core_axis_name="c", subcore_axis_name="s"),
        compiler_params=pltpu.CompilerParams(needs_layout_passes=False),
        scratch_types=[
            pltpu.VMEM((N,), jnp.float32),
            pltpu.VMEM((4096,), jnp.int32),
            pltpu.VMEM((1024,), jnp.int32),
            pltpu.VMEM((CAP,), jnp.int32),
            pltpu.VMEM((CAP,), jnp.int32),
            pltpu.VMEM((CAP,), jnp.int32),
            pltpu.VMEM((CAP,), jnp.int32),
            pltpu.VMEM((CAP,), jnp.int32),
            pltpu.VMEM((CAP,), jnp.int32),
        ],
    )
    return kfn(x)


def _rotl(v, d):
    u = jnp.uint32(d)
    return (v << u) | (v >> jnp.uint32(32 - d))


def _mask_body(x_ref, p_ref, o_ref, *, block_cols):
    i = pl.program_id(0)
    x = x_ref[...]
    p = p_ref[...]
    rows_blk, cols_blk = x.shape
    row = lax.broadcasted_iota(jnp.uint32, (rows_blk, cols_blk), 0)
    col = lax.broadcasted_iota(jnp.uint32, (rows_blk, cols_blk), 1)
    n = row * jnp.uint32(N) + col + jnp.uint32(block_cols) * i.astype(jnp.uint32)
    ks0 = jnp.uint32(0)
    ks1 = jnp.uint32(42)
    ks2 = jnp.uint32(42 ^ 0x1BD11BDA)
    x0 = jnp.full_like(n, ks0)
    x1 = n + ks1

    def rounds(x0, x1, rots):
        for r in rots:
            x0 = x0 + x1
            x1 = _rotl(x1, r)
            x1 = x0 ^ x1
        return x0, x1

    ra = (13, 15, 26, 6)
    rb = (17, 29, 16, 24)
    x0, x1 = rounds(x0, x1, ra)
    x0 += ks1
    x1 += ks2 + jnp.uint32(1)
    x0, x1 = rounds(x0, x1, rb)
    x0 += ks2
    x1 += ks0 + jnp.uint32(2)
    x0, x1 = rounds(x0, x1, ra)
    x0 += ks0
    x1 += ks1 + jnp.uint32(3)
    x0, x1 = rounds(x0, x1, rb)
    x0 += ks1
    x1 += ks2 + jnp.uint32(4)
    x0, x1 = rounds(x0, x1, ra)
    x0 += ks2
    x1 += ks0 + jnp.uint32(5)
    bits = x0 ^ x1

    fb = (bits >> jnp.uint32(9)) | jnp.uint32(0x3F800000)
    u = lax.bitcast_convert_type(fb, jnp.float32) - jnp.float32(1.0)
    keep = u < (jnp.float32(1.0) - p)
    o_ref[...] = jnp.where(keep, x, jnp.float32(0.0))


def _apply_mask(x, prob):
    block_cols = 4096
    return pl.pallas_call(
        functools.partial(_mask_body, block_cols=block_cols),
        grid=(N // block_cols,),
        in_specs=[
            pl.BlockSpec((R, block_cols), lambda i: (0, i)),
            pl.BlockSpec((R, block_cols), lambda i: (0, i)),
        ],
        out_specs=pl.BlockSpec((R, block_cols), lambda i: (0, i)),
        out_shape=jax.ShapeDtypeStruct((R, N), jnp.float32),
    )(x, prob)


def kernel(x):
    new_prob = _sc_topk_replace(x)
    return _apply_mask(x, new_prob)


# speedup vs baseline: 12.5902x; 1.2039x over previous
import math
import functools

import jax
import jax.numpy as jnp
from jax import lax
from jax.experimental import pallas as pl
from jax.experimental.pallas import tpu as pltpu
from jax.experimental.pallas import tpu_sc as plsc

R, N = 128, 32768
K = math.floor(0.1 * N)
NVEC = N // 16
CAP = 8192
NW = 32
ROWS_PER_W = R // NW
SENT_HI = 0x7FFFFFFF


def _lane0(v):
    return lax.squeeze(lax.slice(v, (0,), (1,)), (0,))


def _lane15(v):
    return lax.squeeze(lax.slice(v, (15,), (16,)), (0,))


def _sc_body(x_hbm, out_hbm, P, h1a, h1b, h2a, h2b,
             BA, BB, BAo, TA, TB, TAo, IA, IB, IAo):
    wid = lax.axis_index("s") * 2 + lax.axis_index("c")
    lane = lax.iota(jnp.int32, 16)
    zeros16 = jnp.zeros((16,), jnp.int32)

    czero, _ = plsc.scan_count(zeros16)
    c0 = jnp.min(czero)
    e0 = jnp.int32(1) - c0

    def hist_bump(href, d, cnt, lastm):
        base = plsc.load_gather(href, [d])
        plsc.store_scatter(href, [d], base + cnt + e0, mask=lastm)
        return base

    def clear(href, nv):
        def body(i, _):
            href[pl.ds(i * 16, 16)] = zeros16
            return 0
        lax.fori_loop(0, nv, body, 0)

    def radix_pass(src, dst, shift, nvec, isrc, idst):
        clear(h2a, 64)
        clear(h2b, 64)

        def hist(i, _):
            ue = src[pl.ds(i * 32, 16)]
            uo = src[pl.ds(i * 32 + 16, 16)]
            de = (ue >> shift) & 1023
            do = (uo >> shift) & 1023
            ce, le = plsc.scan_count(de)
            co, lo = plsc.scan_count(do)
            hist_bump(h2a, de, ce, le)
            hist_bump(h2b, do, co, lo)
            return 0
        lax.fori_loop(0, nvec >> 1, hist, 0)

        def csum(i, carry):
            v = h2a[pl.ds(i * 16, 16)] + h2b[pl.ds(i * 16, 16)]
            s = plsc.cumsum(v)
            h2a[pl.ds(i * 16, 16)] = s - v + carry
            return carry + _lane15(s)
        lax.fori_loop(0, 64, csum, jnp.int32(0))

        def scat(i, _):
            u = src[pl.ds(i * 16, 16)]
            d = (u >> shift) & 1023
            cnt, lastm = plsc.scan_count(d)
            base = hist_bump(h2a, d, cnt, lastm)
            off = base + cnt - c0
            plsc.store_scatter(dst, [off], u)
            if isrc is not None:
                ix = isrc[pl.ds(i * 16, 16)]
                plsc.store_scatter(idst, [off], ix)
            return 0
        lax.fori_loop(0, nvec, scat, 0)

    def row_body(rr, _):
        row = wid * ROWS_PER_W + rr
        pltpu.sync_copy(x_hbm.at[row], P)

        clear(h1a, 256)
        clear(h1b, 256)

        def mh(i, carry):
            acce, acco = carry
            ae = jnp.abs(P[pl.ds(i * 32, 16)])
            ao = jnp.abs(P[pl.ds(i * 32 + 16, 16)])
            de = plsc.bitcast(ae, jnp.int32) >> 19
            do = plsc.bitcast(ao, jnp.int32) >> 19
            ce, le = plsc.scan_count(de)
            co, lo = plsc.scan_count(do)
            hist_bump(h1a, de, ce, le)
            hist_bump(h1b, do, co, lo)
            return (jnp.maximum(acce, ae), jnp.maximum(acco, ao))
        z16f = jnp.zeros((16,), jnp.float32)
        acce, acco = lax.fori_loop(0, NVEC // 2, mh, (z16f, z16f))
        m = jnp.max(jnp.maximum(acce, acco))

        def cs1(i, carry):
            c, t1, t2 = carry
            v = h1a[pl.ds(i * 16, 16)] + h1b[pl.ds(i * 16, 16)]
            s = plsc.cumsum(v)
            ex = s - v + c
            t1 = t1 + _lane0(plsc.all_reduce_population_count(ex < K))
            t2 = t2 + _lane0(plsc.all_reduce_population_count(ex <= N - K))
            return (c + _lane15(s), t1, t2)
        _, t1, t2 = lax.fori_loop(
            0, 256, cs1, (jnp.int32(0), jnp.int32(0), jnp.int32(0)))
        H = t2 - 1

        def cp(i, carry):
            pbe, pte, pbo, pto = carry
            sle = pl.ds(i * 32, 16)
            slo = pl.ds(i * 32 + 16, 16)
            ae = jnp.abs(P[sle])
            ao = jnp.abs(P[slo])
            de = plsc.bitcast(ae, jnp.int32) >> 19
            do = plsc.bitcast(ao, jnp.int32) >> 19
            pe = ae / m
            po = ao / m
            P[sle] = pe
            P[slo] = po
            ue = plsc.bitcast(pe, jnp.int32)
            uo = plsc.bitcast(po, jnp.int32)
            mBe = de < t1
            mTe = de >= H
            mBo = do < t1
            mTo = do >= H
            plsc.store_compressed(BA.at[pl.ds(pbe, 16)], ue, mask=mBe)
            plsc.store_compressed(TA.at[pl.ds(pte, 16)], ue, mask=mTe)
            plsc.store_compressed(IA.at[pl.ds(pte, 16)], lane + i * 32,
                                  mask=mTe)
            plsc.store_compressed(BAo.at[pl.ds(pbo, 16)], uo, mask=mBo)
            plsc.store_compressed(TAo.at[pl.ds(pto, 16)], uo, mask=mTo)
            plsc.store_compressed(IAo.at[pl.ds(pto, 16)], lane + i * 32 + 16,
                                  mask=mTo)
            pbe = pbe + _lane0(plsc.all_reduce_population_count(mBe))
            pte = pte + _lane0(plsc.all_reduce_population_count(mTe))
            pbo = pbo + _lane0(plsc.all_reduce_population_count(mBo))
            pto = pto + _lane0(plsc.all_reduce_population_count(mTo))
            return (pbe, pte, pbo, pto)
        z = jnp.int32(0)
        pbe, pte, pbo, pto = lax.fori_loop(0, NVEC // 2, cp, (z, z, z, z))

        def cpyB(i, _):
            BA[pl.ds(pbe + i * 16, 16)] = BAo[pl.ds(i * 16, 16)]
            return 0
        lax.fori_loop(0, (pbo + 15) >> 4, cpyB, 0)

        def cpyT(i, _):
            TA[pl.ds(pte + i * 16, 16)] = TAo[pl.ds(i * 16, 16)]
            IA[pl.ds(pte + i * 16, 16)] = IAo[pl.ds(i * 16, 16)]
            return 0
        lax.fori_loop(0, (pto + 15) >> 4, cpyT, 0)

        posB = pbe + pbo
        posT = pte + pto

        sent = jnp.full((16,), SENT_HI, jnp.int32)
        BA[pl.ds(posB, 16)] = sent
        BA[pl.ds(posB + 16, 16)] = sent
        TA[pl.ds(posT, 16)] = zeros16
        TA[pl.ds(posT + 16, 16)] = zeros16
        IA[pl.ds(posT, 16)] = zeros16
        IA[pl.ds(posT + 16, 16)] = zeros16
        nbB = (((posB + 15) >> 4) + 1) & ~1
        nbT = (((posT + 15) >> 4) + 1) & ~1
        STp = nbT * 16

        radix_pass(BA, BB, 10, nbB, None, None)
        radix_pass(BB, BA, 20, nbB, None, None)

        radix_pass(TA, TB, 0, nbT, IA, IB)
        radix_pass(TB, TA, 10, nbT, IB, IA)
        radix_pass(TA, TB, 20, nbT, IA, IB)

        def rep(i, _):
            t = jnp.minimum(lane + i * 16, K - 1)
            j = STp - K + t
            vu = plsc.load_gather(TB, [j])
            ti = plsc.load_gather(IB, [j])
            bu = plsc.load_gather(BA, [K - 1 - t])
            v = plsc.bitcast(vu, jnp.float32)
            b = plsc.bitcast(bu, jnp.float32)
            plsc.store_scatter(P, [ti], v - (v - b))
            return 0
        lax.fori_loop(0, (K + 15) // 16, rep, 0)

        pltpu.sync_copy(P, out_hbm.at[row])
        return 0

    lax.fori_loop(0, ROWS_PER_W, row_body, 0)


@functools.partial(jax.jit, static_argnums=())
def _sc_topk_replace(x):
    kfn = pl.kernel(
        _sc_body,
        out_type=jax.ShapeDtypeStruct((R, N), jnp.float32),
        mesh=plsc.VectorSubcoreMesh(core_axis_name="c", subcore_axis_name="s"),
        compiler_params=pltpu.CompilerParams(needs_layout_passes=False),
        scratch_types=[
            pltpu.VMEM((N,), jnp.float32),
            pltpu.VMEM((4096,), jnp.int32),
            pltpu.VMEM((4096,), jnp.int32),
            pltpu.VMEM((1024,), jnp.int32),
            pltpu.VMEM((1024,), jnp.int32),
            pltpu.VMEM((CAP,), jnp.int32),
            pltpu.VMEM((CAP,), jnp.int32),
            pltpu.VMEM((CAP // 2,), jnp.int32),
            pltpu.VMEM((CAP,), jnp.int32),
            pltpu.VMEM((CAP,), jnp.int32),
            pltpu.VMEM((CAP // 2,), jnp.int32),
            pltpu.VMEM((CAP,), jnp.int32),
            pltpu.VMEM((CAP,), jnp.int32),
            pltpu.VMEM((CAP // 2,), jnp.int32),
        ],
    )
    return kfn(x)


def _rotl(v, d):
    u = jnp.uint32(d)
    return (v << u) | (v >> jnp.uint32(32 - d))


def _mask_body(x_ref, p_ref, o_ref, *, block_cols):
    i = pl.program_id(0)
    x = x_ref[...]
    p = p_ref[...]
    rows_blk, cols_blk = x.shape
    row = lax.broadcasted_iota(jnp.uint32, (rows_blk, cols_blk), 0)
    col = lax.broadcasted_iota(jnp.uint32, (rows_blk, cols_blk), 1)
    n = row * jnp.uint32(N) + col + jnp.uint32(block_cols) * i.astype(jnp.uint32)
    ks0 = jnp.uint32(0)
    ks1 = jnp.uint32(42)
    ks2 = jnp.uint32(42 ^ 0x1BD11BDA)
    x0 = jnp.full_like(n, ks0)
    x1 = n + ks1

    def rounds(x0, x1, rots):
        for r in rots:
            x0 = x0 + x1
            x1 = _rotl(x1, r)
            x1 = x0 ^ x1
        return x0, x1

    ra = (13, 15, 26, 6)
    rb = (17, 29, 16, 24)
    x0, x1 = rounds(x0, x1, ra)
    x0 += ks1
    x1 += ks2 + jnp.uint32(1)
    x0, x1 = rounds(x0, x1, rb)
    x0 += ks2
    x1 += ks0 + jnp.uint32(2)
    x0, x1 = rounds(x0, x1, ra)
    x0 += ks0
    x1 += ks1 + jnp.uint32(3)
    x0, x1 = rounds(x0, x1, rb)
    x0 += ks1
    x1 += ks2 + jnp.uint32(4)
    x0, x1 = rounds(x0, x1, ra)
    x0 += ks2
    x1 += ks0 + jnp.uint32(5)
    bits = x0 ^ x1

    fb = (bits >> jnp.uint32(9)) | jnp.uint32(0x3F800000)
    u = lax.bitcast_convert_type(fb, jnp.float32) - jnp.float32(1.0)
    keep = u < (jnp.float32(1.0) - p)
    o_ref[...] = jnp.where(keep, x, jnp.float32(0.0))


def _apply_mask(x, prob):
    block_cols = 4096
    return pl.pallas_call(
        functools.partial(_mask_body, block_cols=block_cols),
        grid=(N // block_cols,),
        in_specs=[
            pl.BlockSpec((R, block_cols), lambda i: (0, i)),
            pl.BlockSpec((R, block_cols), lambda i: (0, i)),
        ],
        out_specs=pl.BlockSpec((R, block_cols), lambda i: (0, i)),
        out_shape=jax.ShapeDtypeStruct((R, N), jnp.float32),
    )(x, prob)


def kernel(x):
    new_prob = _sc_topk_replace(x)
    return _apply_mask(x, new_prob)


# speedup vs baseline: 14.2036x; 1.1281x over previous
import math
import functools

import jax
import jax.numpy as jnp
from jax import lax
from jax.experimental import pallas as pl
from jax.experimental.pallas import tpu as pltpu
from jax.experimental.pallas import tpu_sc as plsc

R, N = 128, 32768
K = math.floor(0.1 * N)
NVEC = N // 16
CAP = 4096
NW = 32
ROWS_PER_W = R // NW
SENT_HI = 0x7FFFFFFF


def _lane0(v):
    return lax.squeeze(lax.slice(v, (0,), (1,)), (0,))


def _lane15(v):
    return lax.squeeze(lax.slice(v, (15,), (16,)), (0,))


def _sc_body(x_hbm, out_hbm, P0, P1, h1a, h1b, h2a, h2b,
             BA0, BB0, TA0, TB0, IA0, IB0,
             BA1, BB1, TA1, TB1, IA1, IB1):
    wid = lax.axis_index("s") * 2 + lax.axis_index("c")
    lane = lax.iota(jnp.int32, 16)
    zeros16 = jnp.zeros((16,), jnp.int32)

    czero, _ = plsc.scan_count(zeros16)
    c0 = jnp.min(czero)
    e0 = jnp.int32(1) - c0

    def hist_bump(href, d, cnt, lastm):
        base = plsc.load_gather(href, [d])
        plsc.store_scatter(href, [d], base + cnt + e0, mask=lastm)
        return base

    def clear2(ha, hb, nv):
        def body(i, _):
            ha[pl.ds(i * 16, 16)] = zeros16
            hb[pl.ds(i * 16, 16)] = zeros16
            return 0
        lax.fori_loop(0, nv, body, 0)

    def radix_pass2(shift, s0, d0, is0, id0, nb0, s1, d1, is1, id1, nb1):
        clear2(h2a, h2b, 64)
        nb = jnp.maximum(nb0, nb1)

        def hist(i, _):
            vi = zeros16 + i
            m0 = vi < nb0
            m1 = vi < nb1
            u0 = s0[pl.ds(i * 16, 16)]
            u1 = s1[pl.ds(i * 16, 16)]
            g0 = (u0 >> shift) & 1023
            g1 = (u1 >> shift) & 1023
            c0v, l0v = plsc.scan_count(g0, m0)
            c1v, l1v = plsc.scan_count(g1, m1)
            hist_bump(h2a, g0, c0v, l0v)
            hist_bump(h2b, g1, c1v, l1v)
            return 0
        lax.fori_loop(0, nb, hist, 0)

        def csum(i, carry):
            ca, cb = carry
            va = h2a[pl.ds(i * 16, 16)]
            vb = h2b[pl.ds(i * 16, 16)]
            sa = plsc.cumsum(va)
            sb = plsc.cumsum(vb)
            h2a[pl.ds(i * 16, 16)] = sa - va + ca
            h2b[pl.ds(i * 16, 16)] = sb - vb + cb
            return (ca + _lane15(sa), cb + _lane15(sb))
        lax.fori_loop(0, 64, csum, (jnp.int32(0), jnp.int32(0)))

        def scat(i, _):
            vi = zeros16 + i
            m0 = vi < nb0
            m1 = vi < nb1
            u0 = s0[pl.ds(i * 16, 16)]
            u1 = s1[pl.ds(i * 16, 16)]
            g0 = (u0 >> shift) & 1023
            g1 = (u1 >> shift) & 1023
            c0v, l0v = plsc.scan_count(g0, m0)
            c1v, l1v = plsc.scan_count(g1, m1)
            b0 = hist_bump(h2a, g0, c0v, l0v)
            b1 = hist_bump(h2b, g1, c1v, l1v)
            o0 = b0 + c0v - c0
            o1 = b1 + c1v - c0
            plsc.store_scatter(d0, [o0], u0, mask=m0)
            plsc.store_scatter(d1, [o1], u1, mask=m1)
            if is0 is not None:
                plsc.store_scatter(id0, [o0], is0[pl.ds(i * 16, 16)], mask=m0)
                plsc.store_scatter(id1, [o1], is1[pl.ds(i * 16, 16)], mask=m1)
            return 0
        lax.fori_loop(0, nb, scat, 0)

    def pair_body(pp, _):
        row0 = wid * ROWS_PER_W + pp * 2
        row1 = row0 + 1
        pltpu.sync_copy(x_hbm.at[row0], P0)
        pltpu.sync_copy(x_hbm.at[row1], P1)

        clear2(h1a, h1b, 256)

        def mh(i, carry):
            acc0, acc1 = carry
            a0 = jnp.abs(P0[pl.ds(i * 16, 16)])
            a1 = jnp.abs(P1[pl.ds(i * 16, 16)])
            g0 = plsc.bitcast(a0, jnp.int32) >> 19
            g1 = plsc.bitcast(a1, jnp.int32) >> 19
            c0v, l0v = plsc.scan_count(g0)
            c1v, l1v = plsc.scan_count(g1)
            hist_bump(h1a, g0, c0v, l0v)
            hist_bump(h1b, g1, c1v, l1v)
            return (jnp.maximum(acc0, a0), jnp.maximum(acc1, a1))
        z16f = jnp.zeros((16,), jnp.float32)
        acc0, acc1 = lax.fori_loop(0, NVEC, mh, (z16f, z16f))
        m0 = jnp.max(acc0)
        m1 = jnp.max(acc1)

        def cs1(i, carry):
            ca, t1a, t2a, cb, t1b, t2b = carry
            va = h1a[pl.ds(i * 16, 16)]
            vb = h1b[pl.ds(i * 16, 16)]
            sa = plsc.cumsum(va)
            sb = plsc.cumsum(vb)
            exa = sa - va + ca
            exb = sb - vb + cb
            t1a = t1a + _lane0(plsc.all_reduce_population_count(exa < K))
            t2a = t2a + _lane0(plsc.all_reduce_population_count(exa <= N - K))
            t1b = t1b + _lane0(plsc.all_reduce_population_count(exb < K))
            t2b = t2b + _lane0(plsc.all_reduce_population_count(exb <= N - K))
            return (ca + _lane15(sa), t1a, t2a, cb + _lane15(sb), t1b, t2b)
        z = jnp.int32(0)
        _, t1_0, t2_0, _, t1_1, t2_1 = lax.fori_loop(
            0, 256, cs1, (z, z, z, z, z, z))
        H0 = t2_0 - 1
        H1 = t2_1 - 1

        def cp(i, carry):
            pb0, pt0, pb1, pt1 = carry
            sl = pl.ds(i * 16, 16)
            a0 = jnp.abs(P0[sl])
            a1 = jnp.abs(P1[sl])
            g0 = plsc.bitcast(a0, jnp.int32) >> 19
            g1 = plsc.bitcast(a1, jnp.int32) >> 19
            p0 = a0 / m0
            p1 = a1 / m1
            P0[sl] = p0
            P1[sl] = p1
            u0 = plsc.bitcast(p0, jnp.int32)
            u1 = plsc.bitcast(p1, jnp.int32)
            mB0 = g0 < t1_0
            mT0 = g0 >= H0
            mB1 = g1 < t1_1
            mT1 = g1 >= H1
            ix = lane + i * 16
            plsc.store_compressed(BA0.at[pl.ds(pb0, 16)], u0, mask=mB0)
            plsc.store_compressed(TA0.at[pl.ds(pt0, 16)], u0, mask=mT0)
            plsc.store_compressed(IA0.at[pl.ds(pt0, 16)], ix, mask=mT0)
            plsc.store_compressed(BA1.at[pl.ds(pb1, 16)], u1, mask=mB1)
            plsc.store_compressed(TA1.at[pl.ds(pt1, 16)], u1, mask=mT1)
            plsc.store_compressed(IA1.at[pl.ds(pt1, 16)], ix, mask=mT1)
            pb0 = pb0 + _lane0(plsc.all_reduce_population_count(mB0))
            pt0 = pt0 + _lane0(plsc.all_reduce_population_count(mT0))
            pb1 = pb1 + _lane0(plsc.all_reduce_population_count(mB1))
            pt1 = pt1 + _lane0(plsc.all_reduce_population_count(mT1))
            return (pb0, pt0, pb1, pt1)
        pb0, pt0, pb1, pt1 = lax.fori_loop(0, NVEC, cp, (z, z, z, z))

        sent = jnp.full((16,), SENT_HI, jnp.int32)
        BA0[pl.ds(pb0, 16)] = sent
        BA1[pl.ds(pb1, 16)] = sent
        TA0[pl.ds(pt0, 16)] = zeros16
        TA1[pl.ds(pt1, 16)] = zeros16
        IA0[pl.ds(pt0, 16)] = zeros16
        IA1[pl.ds(pt1, 16)] = zeros16
        nbB0 = (pb0 + 15) >> 4
        nbB1 = (pb1 + 15) >> 4
        nbT0 = (pt0 + 15) >> 4
        nbT1 = (pt1 + 15) >> 4
        STp0 = nbT0 * 16
        STp1 = nbT1 * 16

        radix_pass2(10, BA0, BB0, None, None, nbB0, BA1, BB1, None, None, nbB1)
        radix_pass2(20, BB0, BA0, None, None, nbB0, BB1, BA1, None, None, nbB1)

        radix_pass2(0, TA0, TB0, IA0, IB0, nbT0, TA1, TB1, IA1, IB1, nbT1)
        radix_pass2(10, TB0, TA0, IB0, IA0, nbT0, TB1, TA1, IB1, IA1, nbT1)
        radix_pass2(20, TA0, TB0, IA0, IB0, nbT0, TA1, TB1, IA1, IB1, nbT1)

        def rep(i, _):
            t = jnp.minimum(lane + i * 16, K - 1)
            j0 = STp0 - K + t
            j1 = STp1 - K + t
            vu0 = plsc.load_gather(TB0, [j0])
            ti0 = plsc.load_gather(IB0, [j0])
            bu0 = plsc.load_gather(BA0, [K - 1 - t])
            vu1 = plsc.load_gather(TB1, [j1])
            ti1 = plsc.load_gather(IB1, [j1])
            bu1 = plsc.load_gather(BA1, [K - 1 - t])
            v0 = plsc.bitcast(vu0, jnp.float32)
            b0 = plsc.bitcast(bu0, jnp.float32)
            v1 = plsc.bitcast(vu1, jnp.float32)
            b1 = plsc.bitcast(bu1, jnp.float32)
            plsc.store_scatter(P0, [ti0], v0 - (v0 - b0))
            plsc.store_scatter(P1, [ti1], v1 - (v1 - b1))
            return 0
        lax.fori_loop(0, (K + 15) // 16, rep, 0)

        pltpu.sync_copy(P0, out_hbm.at[row0])
        pltpu.sync_copy(P1, out_hbm.at[row1])
        return 0

    lax.fori_loop(0, ROWS_PER_W // 2, pair_body, 0)


@functools.partial(jax.jit, static_argnums=())
def _sc_topk_replace(x):
    row_scratch = []
    for _ in range(2):
        row_scratch += [pltpu.VMEM((CAP,), jnp.int32) for _ in range(6)]
    kfn = pl.kernel(
        _sc_body,
        out_type=jax.ShapeDtypeStruct((R, N), jnp.float32),
        mesh=plsc.VectorSubcoreMesh(core_axis_name="c", subcore_axis_name="s"),
        compiler_params=pltpu.CompilerParams(needs_layout_passes=False),
        scratch_types=[
            pltpu.VMEM((N,), jnp.float32),
            pltpu.VMEM((N,), jnp.float32),
            pltpu.VMEM((4096,), jnp.int32),
            pltpu.VMEM((4096,), jnp.int32),
            pltpu.VMEM((1024,), jnp.int32),
            pltpu.VMEM((1024,), jnp.int32),
        ] + row_scratch,
    )
    return kfn(x)


def _rotl(v, d):
    u = jnp.uint32(d)
    return (v << u) | (v >> jnp.uint32(32 - d))


def _mask_body(x_ref, p_ref, o_ref, *, block_cols):
    i = pl.program_id(0)
    x = x_ref[...]
    p = p_ref[...]
    rows_blk, cols_blk = x.shape
    row = lax.broadcasted_iota(jnp.uint32, (rows_blk, cols_blk), 0)
    col = lax.broadcasted_iota(jnp.uint32, (rows_blk, cols_blk), 1)
    n = row * jnp.uint32(N) + col + jnp.uint32(block_cols) * i.astype(jnp.uint32)
    ks0 = jnp.uint32(0)
    ks1 = jnp.uint32(42)
    ks2 = jnp.uint32(42 ^ 0x1BD11BDA)
    x0 = jnp.full_like(n, ks0)
    x1 = n + ks1

    def rounds(x0, x1, rots):
        for r in rots:
            x0 = x0 + x1
            x1 = _rotl(x1, r)
            x1 = x0 ^ x1
        return x0, x1

    ra = (13, 15, 26, 6)
    rb = (17, 29, 16, 24)
    x0, x1 = rounds(x0, x1, ra)
    x0 += ks1
    x1 += ks2 + jnp.uint32(1)
    x0, x1 = rounds(x0, x1, rb)
    x0 += ks2
    x1 += ks0 + jnp.uint32(2)
    x0, x1 = rounds(x0, x1, ra)
    x0 += ks0
    x1 += ks1 + jnp.uint32(3)
    x0, x1 = rounds(x0, x1, rb)
    x0 += ks1
    x1 += ks2 + jnp.uint32(4)
    x0, x1 = rounds(x0, x1, ra)
    x0 += ks2
    x1 += ks0 + jnp.uint32(5)
    bits = x0 ^ x1

    fb = (bits >> jnp.uint32(9)) | jnp.uint32(0x3F800000)
    u = lax.bitcast_convert_type(fb, jnp.float32) - jnp.float32(1.0)
    keep = u < (jnp.float32(1.0) - p)
    o_ref[...] = jnp.where(keep, x, jnp.float32(0.0))


def _apply_mask(x, prob):
    block_cols = 4096
    return pl.pallas_call(
        functools.partial(_mask_body, block_cols=block_cols),
        grid=(N // block_cols,),
        in_specs=[
            pl.BlockSpec((R, block_cols), lambda i: (0, i)),
            pl.BlockSpec((R, block_cols), lambda i: (0, i)),
        ],
        out_specs=pl.BlockSpec((R, block_cols), lambda i: (0, i)),
        out_shape=jax.ShapeDtypeStruct((R, N), jnp.float32),
    )(x, prob)


def kernel(x):
    new_prob = _sc_topk_replace(x)
    return _apply_mask(x, new_prob)


# speedup vs baseline: 15.4624x; 1.0886x over previous
import math
import functools

import jax
import jax.numpy as jnp
from jax import lax
from jax.experimental import pallas as pl
from jax.experimental.pallas import tpu as pltpu
from jax.experimental.pallas import tpu_sc as plsc

R, N = 128, 32768
K = math.floor(0.1 * N)
NVEC = N // 16
CAP = 4096
NW = 32
ROWS_PER_W = R // NW
SENT_HI = 0x7FFFFFFF


def _lane0(v):
    return lax.squeeze(lax.slice(v, (0,), (1,)), (0,))


def _lane15(v):
    return lax.squeeze(lax.slice(v, (15,), (16,)), (0,))


def _sc_body(x_hbm, out_hbm, P0, P1, h1a, h1b, h2a, h2b,
             BA0, BB0, TA0, TB0, IA0, IB0,
             BA1, BB1, TA1, TB1, IA1, IB1):
    wid = lax.axis_index("s") * 2 + lax.axis_index("c")
    lane = lax.iota(jnp.int32, 16)
    zeros16 = jnp.zeros((16,), jnp.int32)

    czero, _ = plsc.scan_count(zeros16)
    c0 = jnp.min(czero)
    e0 = jnp.int32(1) - c0

    def hist_bump(href, d, cnt, lastm):
        plsc.addupdate_scatter(href, [d], cnt + e0, mask=lastm)

    def rank_bump(href, d, cnt, lastm):
        base = plsc.load_gather(href, [d])
        plsc.store_scatter(href, [d], base + cnt + e0, mask=lastm)
        return base

    def clear2(ha, hb, nv):
        def body(i, _):
            ha[pl.ds(i * 16, 16)] = zeros16
            hb[pl.ds(i * 16, 16)] = zeros16
            return 0
        lax.fori_loop(0, nv, body, 0)

    def radix_pass2(shift, s0, d0, is0, id0, nb0, s1, d1, is1, id1, nb1):
        clear2(h2a, h2b, 64)
        nb = jnp.maximum(nb0, nb1)

        def hist(i, _):
            vi = zeros16 + i
            m0 = vi < nb0
            m1 = vi < nb1
            u0 = s0[pl.ds(i * 16, 16)]
            u1 = s1[pl.ds(i * 16, 16)]
            g0 = (u0 >> shift) & 1023
            g1 = (u1 >> shift) & 1023
            c0v, l0v = plsc.scan_count(g0, m0)
            c1v, l1v = plsc.scan_count(g1, m1)
            hist_bump(h2a, g0, c0v, l0v)
            hist_bump(h2b, g1, c1v, l1v)
            return 0
        lax.fori_loop(0, nb, hist, 0)

        def csum(i, carry):
            ca, cb = carry
            va = h2a[pl.ds(i * 16, 16)]
            vb = h2b[pl.ds(i * 16, 16)]
            sa = plsc.cumsum(va)
            sb = plsc.cumsum(vb)
            h2a[pl.ds(i * 16, 16)] = sa - va + ca
            h2b[pl.ds(i * 16, 16)] = sb - vb + cb
            return (ca + _lane15(sa), cb + _lane15(sb))
        lax.fori_loop(0, 64, csum, (jnp.int32(0), jnp.int32(0)))

        def scat(i, _):
            vi = zeros16 + i
            m0 = vi < nb0
            m1 = vi < nb1
            u0 = s0[pl.ds(i * 16, 16)]
            u1 = s1[pl.ds(i * 16, 16)]
            g0 = (u0 >> shift) & 1023
            g1 = (u1 >> shift) & 1023
            c0v, l0v = plsc.scan_count(g0, m0)
            c1v, l1v = plsc.scan_count(g1, m1)
            b0 = rank_bump(h2a, g0, c0v, l0v)
            b1 = rank_bump(h2b, g1, c1v, l1v)
            o0 = b0 + c0v - c0
            o1 = b1 + c1v - c0
            plsc.store_scatter(d0, [o0], u0, mask=m0)
            plsc.store_scatter(d1, [o1], u1, mask=m1)
            if is0 is not None:
                plsc.store_scatter(id0, [o0], is0[pl.ds(i * 16, 16)], mask=m0)
                plsc.store_scatter(id1, [o1], is1[pl.ds(i * 16, 16)], mask=m1)
            return 0
        lax.fori_loop(0, nb, scat, 0)

    def pair_body(pp, _):
        row0 = wid * ROWS_PER_W + pp * 2
        row1 = row0 + 1
        pltpu.sync_copy(x_hbm.at[row0], P0)
        pltpu.sync_copy(x_hbm.at[row1], P1)

        clear2(h1a, h1b, 256)

        def mh(i, carry):
            acc0, acc1 = carry
            a0 = jnp.abs(P0[pl.ds(i * 16, 16)])
            a1 = jnp.abs(P1[pl.ds(i * 16, 16)])
            g0 = plsc.bitcast(a0, jnp.int32) >> 19
            g1 = plsc.bitcast(a1, jnp.int32) >> 19
            c0v, l0v = plsc.scan_count(g0)
            c1v, l1v = plsc.scan_count(g1)
            hist_bump(h1a, g0, c0v, l0v)
            hist_bump(h1b, g1, c1v, l1v)
            return (jnp.maximum(acc0, a0), jnp.maximum(acc1, a1))
        z16f = jnp.zeros((16,), jnp.float32)
        acc0, acc1 = lax.fori_loop(0, NVEC, mh, (z16f, z16f))
        m0 = jnp.max(acc0)
        m1 = jnp.max(acc1)
        r0 = jnp.float32(1.0) / (jnp.zeros((16,), jnp.float32) + m0)
        r1 = jnp.float32(1.0) / (jnp.zeros((16,), jnp.float32) + m1)

        def cs1(i, carry):
            ca, t1a, t2a, cb, t1b, t2b = carry
            va = h1a[pl.ds(i * 16, 16)]
            vb = h1b[pl.ds(i * 16, 16)]
            sa = plsc.cumsum(va)
            sb = plsc.cumsum(vb)
            exa = sa - va + ca
            exb = sb - vb + cb
            t1a = t1a + _lane0(plsc.all_reduce_population_count(exa < K))
            t2a = t2a + _lane0(plsc.all_reduce_population_count(exa <= N - K))
            t1b = t1b + _lane0(plsc.all_reduce_population_count(exb < K))
            t2b = t2b + _lane0(plsc.all_reduce_population_count(exb <= N - K))
            return (ca + _lane15(sa), t1a, t2a, cb + _lane15(sb), t1b, t2b)
        z = jnp.int32(0)
        _, t1_0, t2_0, _, t1_1, t2_1 = lax.fori_loop(
            0, 256, cs1, (z, z, z, z, z, z))
        H0 = t2_0 - 1
        H1 = t2_1 - 1

        def cp(i, carry):
            pb0, pt0, pb1, pt1 = carry
            sl = pl.ds(i * 16, 16)
            a0 = jnp.abs(P0[sl])
            a1 = jnp.abs(P1[sl])
            g0 = plsc.bitcast(a0, jnp.int32) >> 19
            g1 = plsc.bitcast(a1, jnp.int32) >> 19
            p0 = a0 * r0
            p1 = a1 * r1
            P0[sl] = p0
            P1[sl] = p1
            u0 = plsc.bitcast(p0, jnp.int32)
            u1 = plsc.bitcast(p1, jnp.int32)
            mB0 = g0 < t1_0
            mT0 = g0 >= H0
            mB1 = g1 < t1_1
            mT1 = g1 >= H1
            ix = lane + i * 16
            plsc.store_compressed(BA0.at[pl.ds(pb0, 16)], u0, mask=mB0)
            plsc.store_compressed(TA0.at[pl.ds(pt0, 16)], u0, mask=mT0)
            plsc.store_compressed(IA0.at[pl.ds(pt0, 16)], ix, mask=mT0)
            plsc.store_compressed(BA1.at[pl.ds(pb1, 16)], u1, mask=mB1)
            plsc.store_compressed(TA1.at[pl.ds(pt1, 16)], u1, mask=mT1)
            plsc.store_compressed(IA1.at[pl.ds(pt1, 16)], ix, mask=mT1)
            pb0 = pb0 + _lane0(plsc.all_reduce_population_count(mB0))
            pt0 = pt0 + _lane0(plsc.all_reduce_population_count(mT0))
            pb1 = pb1 + _lane0(plsc.all_reduce_population_count(mB1))
            pt1 = pt1 + _lane0(plsc.all_reduce_population_count(mT1))
            return (pb0, pt0, pb1, pt1)
        pb0, pt0, pb1, pt1 = lax.fori_loop(0, NVEC, cp, (z, z, z, z))

        sent = jnp.full((16,), SENT_HI, jnp.int32)
        BA0[pl.ds(pb0, 16)] = sent
        BA1[pl.ds(pb1, 16)] = sent
        TA0[pl.ds(pt0, 16)] = zeros16
        TA1[pl.ds(pt1, 16)] = zeros16
        IA0[pl.ds(pt0, 16)] = zeros16
        IA1[pl.ds(pt1, 16)] = zeros16
        nbB0 = (pb0 + 15) >> 4
        nbB1 = (pb1 + 15) >> 4
        nbT0 = (pt0 + 15) >> 4
        nbT1 = (pt1 + 15) >> 4
        STp0 = nbT0 * 16
        STp1 = nbT1 * 16

        radix_pass2(10, BA0, BB0, None, None, nbB0, BA1, BB1, None, None, nbB1)
        radix_pass2(20, BB0, BA0, None, None, nbB0, BB1, BA1, None, None, nbB1)

        radix_pass2(0, TA0, TB0, IA0, IB0, nbT0, TA1, TB1, IA1, IB1, nbT1)
        radix_pass2(10, TB0, TA0, IB0, IA0, nbT0, TB1, TA1, IB1, IA1, nbT1)
        radix_pass2(20, TA0, TB0, IA0, IB0, nbT0, TA1, TB1, IA1, IB1, nbT1)

        def rep(i, _):
            t = jnp.minimum(lane + i * 16, K - 1)
            j0 = STp0 - K + t
            j1 = STp1 - K + t
            vu0 = plsc.load_gather(TB0, [j0])
            ti0 = plsc.load_gather(IB0, [j0])
            bu0 = plsc.load_gather(BA0, [K - 1 - t])
            vu1 = plsc.load_gather(TB1, [j1])
            ti1 = plsc.load_gather(IB1, [j1])
            bu1 = plsc.load_gather(BA1, [K - 1 - t])
            v0 = plsc.bitcast(vu0, jnp.float32)
            b0 = plsc.bitcast(bu0, jnp.float32)
            v1 = plsc.bitcast(vu1, jnp.float32)
            b1 = plsc.bitcast(bu1, jnp.float32)
            plsc.store_scatter(P0, [ti0], v0 - (v0 - b0))
            plsc.store_scatter(P1, [ti1], v1 - (v1 - b1))
            return 0
        lax.fori_loop(0, (K + 15) // 16, rep, 0)

        pltpu.sync_copy(P0, out_hbm.at[row0])
        pltpu.sync_copy(P1, out_hbm.at[row1])
        return 0

    lax.fori_loop(0, ROWS_PER_W // 2, pair_body, 0)


@functools.partial(jax.jit, static_argnums=())
def _sc_topk_replace(x):
    row_scratch = []
    for _ in range(2):
        row_scratch += [pltpu.VMEM((CAP,), jnp.int32) for _ in range(6)]
    kfn = pl.kernel(
        _sc_body,
        out_type=jax.ShapeDtypeStruct((R, N), jnp.float32),
        mesh=plsc.VectorSubcoreMesh(core_axis_name="c", subcore_axis_name="s"),
        compiler_params=pltpu.CompilerParams(needs_layout_passes=False),
        scratch_types=[
            pltpu.VMEM((N,), jnp.float32),
            pltpu.VMEM((N,), jnp.float32),
            pltpu.VMEM((4096,), jnp.int32),
            pltpu.VMEM((4096,), jnp.int32),
            pltpu.VMEM((1024,), jnp.int32),
            pltpu.VMEM((1024,), jnp.int32),
        ] + row_scratch,
    )
    return kfn(x)


def _rotl(v, d):
    u = jnp.uint32(d)
    return (v << u) | (v >> jnp.uint32(32 - d))


def _mask_body(x_ref, p_ref, o_ref, *, block_cols):
    i = pl.program_id(0)
    x = x_ref[...]
    p = p_ref[...]
    rows_blk, cols_blk = x.shape
    row = lax.broadcasted_iota(jnp.uint32, (rows_blk, cols_blk), 0)
    col = lax.broadcasted_iota(jnp.uint32, (rows_blk, cols_blk), 1)
    n = row * jnp.uint32(N) + col + jnp.uint32(block_cols) * i.astype(jnp.uint32)
    ks0 = jnp.uint32(0)
    ks1 = jnp.uint32(42)
    ks2 = jnp.uint32(42 ^ 0x1BD11BDA)
    x0 = jnp.full_like(n, ks0)
    x1 = n + ks1

    def rounds(x0, x1, rots):
        for r in rots:
            x0 = x0 + x1
            x1 = _rotl(x1, r)
            x1 = x0 ^ x1
        return x0, x1

    ra = (13, 15, 26, 6)
    rb = (17, 29, 16, 24)
    x0, x1 = rounds(x0, x1, ra)
    x0 += ks1
    x1 += ks2 + jnp.uint32(1)
    x0, x1 = rounds(x0, x1, rb)
    x0 += ks2
    x1 += ks0 + jnp.uint32(2)
    x0, x1 = rounds(x0, x1, ra)
    x0 += ks0
    x1 += ks1 + jnp.uint32(3)
    x0, x1 = rounds(x0, x1, rb)
    x0 += ks1
    x1 += ks2 + jnp.uint32(4)
    x0, x1 = rounds(x0, x1, ra)
    x0 += ks2
    x1 += ks0 + jnp.uint32(5)
    bits = x0 ^ x1

    fb = (bits >> jnp.uint32(9)) | jnp.uint32(0x3F800000)
    u = lax.bitcast_convert_type(fb, jnp.float32) - jnp.float32(1.0)
    keep = u < (jnp.float32(1.0) - p)
    o_ref[...] = jnp.where(keep, x, jnp.float32(0.0))


def _apply_mask(x, prob):
    block_cols = 4096
    return pl.pallas_call(
        functools.partial(_mask_body, block_cols=block_cols),
        grid=(N // block_cols,),
        in_specs=[
            pl.BlockSpec((R, block_cols), lambda i: (0, i)),
            pl.BlockSpec((R, block_cols), lambda i: (0, i)),
        ],
        out_specs=pl.BlockSpec((R, block_cols), lambda i: (0, i)),
        out_shape=jax.ShapeDtypeStruct((R, N), jnp.float32),
    )(x, prob)


def kernel(x):
    new_prob = _sc_topk_replace(x)
    return _apply_mask(x, new_prob)
